# Initial kernel scaffold; baseline (speedup 1.0000x reference)
import jax
import jax.numpy as jnp
from jax.experimental import pallas as pl


def kernel(x, W_in, b_in, ln_scale, ln_bias, ff_w, ff_b, ff_mask, W_out, b_out):
    raise NotImplementedError("write your pallas kernel here")


# speedup vs baseline: 14.8563x; 14.8563x over previous
import functools

import jax
import jax.numpy as jnp
from jax.experimental import pallas as pl
from jax.experimental.pallas import tpu as pltpu

_LN_EPS = 1e-5
_INV_SQRT2 = 0.7071067811865476

_ERF_CLAMP = 3.832506856900711
_ERF_ALPHA = (0.00022905065861350646, 0.0034082910107109506,
              0.050955695062380861, 0.18520832239976145, 1.128379143519084)
_ERF_BETA = (-1.1791602954361697e-07, 2.3547966471313185e-05,
             0.0010179625278914885, 0.014070470171167667,
             0.11098505178285362, 0.49746925110067538, 1.0)


def _erf(x):
    x = jnp.clip(x, -_ERF_CLAMP, _ERF_CLAMP)
    t = x * x
    num = jnp.float32(_ERF_ALPHA[0])
    for c in _ERF_ALPHA[1:]:
        num = num * t + jnp.float32(c)
    den = jnp.float32(_ERF_BETA[0])
    for c in _ERF_BETA[1:]:
        den = den * t + jnp.float32(c)
    return x * num / den


def _row_count(mask):
    return jnp.sum(mask.astype(jnp.int32), axis=1, keepdims=True)


def _topk_keep(a, k):
    bb, n = a.shape
    key = jax.lax.bitcast_convert_type(a, jnp.int32)
    key = jnp.where(key < 0, key ^ jnp.int32(0x7FFFFFFF), key)

    lo0 = jnp.full((bb, 1), jnp.iinfo(jnp.int32).min, jnp.int32)
    hi0 = jnp.full((bb, 1), jnp.iinfo(jnp.int32).max, jnp.int32)

    def vstep(_, carry):
        lo, hi = carry
        xh = lo ^ hi
        mid = (lo & hi) + (xh >> 1) + (xh & 1)
        cnt = _row_count(key >= mid)
        ge = cnt >= k
        return jnp.where(ge, mid, lo), jnp.where(ge, hi, mid - 1)

    t, _ = jax.lax.fori_loop(0, 32, vstep, (lo0, hi0))

    gt = key > t
    eq = key == t
    m = k - _row_count(gt)
    iota = jax.lax.broadcasted_iota(jnp.int32, (bb, n), 1)

    clo0 = jnp.zeros((bb, 1), jnp.int32)
    chi0 = jnp.full((bb, 1), n, jnp.int32)

    def istep(_, carry):
        clo, chi = carry
        cmid = (clo + chi) >> 1
        cc = _row_count(eq & (iota < cmid))
        geq = cc >= m
        return jnp.where(geq, clo, cmid + 1), jnp.where(geq, cmid, chi)

    _, c = jax.lax.fori_loop(0, 12, istep, (clo0, chi0))
    return gt | (eq & (iota < c))


def _population(h, g, b, k):
    mu = jnp.mean(h, axis=1, keepdims=True)
    d = h - mu
    var = jnp.mean(d * d, axis=1, keepdims=True)
    hn = d * jax.lax.rsqrt(var + _LN_EPS) * g + b
    a = hn * (_erf(hn * _INV_SQRT2) + 1.0) * 0.5
    keep = _topk_keep(a, k)
    return jnp.where(keep, a, 0.0)


def _mask_cast_body(w_ref, m_ref, o_ref):
    o_ref[...] = (w_ref[...] * m_ref[...]).astype(jnp.bfloat16)


def _column_body(x_ref, w_in_ref, b_in_ref, g_ref, bt_ref, ffm_ref, ffb_ref,
                 w_out_ref, b_out_ref, o_ref, *, k, nlayers):
    dn = (((1,), (1,)), ((), ()))
    h = jax.lax.dot_general(x_ref[...].astype(jnp.bfloat16), w_in_ref[...],
                            dn, preferred_element_type=jnp.float32)
    h = h + b_in_ref[...]
    for l in range(nlayers - 1):
        a = _population(h, g_ref[l:l + 1, :], bt_ref[l:l + 1, :], k)
        h = jax.lax.dot_general(a.astype(jnp.bfloat16), ffm_ref[l], dn,
                                preferred_element_type=jnp.float32)
        h = h + ffb_ref[l:l + 1, :]
    a = _population(h, g_ref[nlayers - 1:nlayers, :],
                    bt_ref[nlayers - 1:nlayers, :], k)
    o_ref[...] = jax.lax.dot_general(a.astype(jnp.bfloat16), w_out_ref[...],
                                     dn, preferred_element_type=jnp.float32) \
        + b_out_ref[...]


def kernel(x, W_in, b_in, ln_scale, ln_bias, ff_w, ff_b, ff_mask, W_out,
           b_out):
    B, D = x.shape
    N = W_in.shape[0]
    L = ln_scale.shape[0]
    Lm = ff_w.shape[0]
    k = max(1, int(0.1 * N))
    Bb = 256
    RB = 256

    ffm = pl.pallas_call(
        _mask_cast_body,
        grid=(Lm, N // RB),
        in_specs=[
            pl.BlockSpec((1, RB, N), lambda l, r: (l, r, 0)),
            pl.BlockSpec((1, RB, N), lambda l, r: (l, r, 0)),
        ],
        out_specs=pl.BlockSpec((1, RB, N), lambda l, r: (l, r, 0)),
        out_shape=jax.ShapeDtypeStruct((Lm, N, N), jnp.bfloat16),
        compiler_params=pltpu.CompilerParams(
            dimension_semantics=("parallel", "parallel")),
    )(ff_w, ff_mask)

    return pl.pallas_call(
        functools.partial(_column_body, k=k, nlayers=L),
        grid=(B // Bb,),
        in_specs=[
            pl.BlockSpec((Bb, D), lambda i: (i, 0)),
            pl.BlockSpec((N, D), lambda i: (0, 0)),
            pl.BlockSpec((1, N), lambda i: (0, 0)),
            pl.BlockSpec((L, N), lambda i: (0, 0)),
            pl.BlockSpec((L, N), lambda i: (0, 0)),
            pl.BlockSpec((Lm, N, N), lambda i: (0, 0, 0)),
            pl.BlockSpec((Lm, N), lambda i: (0, 0)),
            pl.BlockSpec((N, N), lambda i: (0, 0)),
            pl.BlockSpec((1, N), lambda i: (0, 0)),
        ],
        out_specs=pl.BlockSpec((Bb, N), lambda i: (i, 0)),
        out_shape=jax.ShapeDtypeStruct((B, N), jnp.float32),
        compiler_params=pltpu.CompilerParams(
            dimension_semantics=("parallel",),
            vmem_limit_bytes=100 * 1024 * 1024,
        ),
    )(x, W_in.astype(jnp.bfloat16), b_in.reshape(1, N), ln_scale, ln_bias,
      ffm, ff_b, W_out.astype(jnp.bfloat16), b_out.reshape(1, N))
